# baseline (speedup 1.0000x reference)
import numpy as np
import jax
import jax.numpy as jnp
from jax import lax
from jax.experimental import pallas as pl
from jax.experimental.pallas import tpu as pltpu
from jax.experimental.pallas import tpu_sc as plsc

N_NODES = 10000
N_EDGES = 320000
N_HEADS = 8
CH = 32
D = 128
DW = D + 16
B = 40
NCHUNK = N_EDGES // B
NC = 2
NS = 16
N_TILES = NC * NS
NPAD = 10240
ROWS_PER_TILE = NPAD // NS
ST = 40


def _sc_body(k_hbm, v_hbm, q_hbm, dst_hbm, z_hbm, rows_hbm,
             accp_hbm, idx_v, qg, kbuf, vbuf, msg, acc_sh,
             sem, semk, semv, semq):
    cid = lax.axis_index("c")
    sid = lax.axis_index("s")
    wid = sid * NC + cid
    r0 = sid * ROWS_PER_TILE
    n_st = ROWS_PER_TILE // ST

    def wrow(j):
        return pl.ds(r0 + ST * j, ST)

    pltpu.sync_copy(z_hbm, msg)
    for j in range(n_st):
        pltpu.sync_copy(rows_hbm.at[wrow(j)], idx_v)
        pltpu.sync_copy(msg, acc_sh.at[idx_v])
    plsc.subcore_barrier()

    lanes = lax.iota(jnp.int32, 16)
    head_mask = lanes < N_HEADS

    n_iter = NCHUNK // N_TILES

    def chunk_body(i, carry):
        cidx = pltpu.async_copy(dst_hbm.at[wid, i, np.int32(0)], idx_v, sem)
        ck = pltpu.async_copy(k_hbm.at[wid, i], kbuf, semk)
        cv = pltpu.async_copy(v_hbm.at[wid, i], vbuf, semv)
        cidx.wait()
        cq = pltpu.async_copy(q_hbm.at[idx_v], qg, semq)
        ck.wait()
        cv.wait()
        cq.wait()

        def edge_body(e, carry2):
            acc = jnp.zeros((16,), jnp.float32)
            for h in range(N_HEADS):
                kv = kbuf[e, pl.ds(16 * h, 16)]
                qv = qg[e, pl.ds(16 * h, 16)]
                acc = jnp.where(lanes == h, jnp.sum(kv * qv), acc)
            w = jnp.where(head_mask, jnp.exp(acc), 0.0)
            for h in range(N_HEADS):
                msg[e, pl.ds(16 * h, 16)] = vbuf[e, pl.ds(16 * h, 16)] * w[h]
            msg[e, pl.ds(D, 16)] = w
            return carry2

        lax.fori_loop(np.int32(0), np.int32(B), edge_body, np.int32(0),
                      unroll=8)

        pltpu.sync_copy(msg, acc_sh.at[idx_v], add=True)
        return carry

    lax.fori_loop(np.int32(0), np.int32(n_iter), chunk_body, np.int32(0))

    plsc.subcore_barrier()
    for j in range(n_st):
        pltpu.sync_copy(rows_hbm.at[wrow(j)], idx_v)
        pltpu.async_copy(acc_sh.at[idx_v], msg, sem).wait()
        pltpu.sync_copy(msg, accp_hbm.at[cid, wrow(j)])


def _sc_edge_pass(k_hm, v_hm, q_hm, dst4):
    mesh = plsc.VectorSubcoreMesh(core_axis_name="c", subcore_axis_name="s",
                                  num_cores=NC, num_subcores=NS)
    f = pl.kernel(
        _sc_body,
        out_type=jax.ShapeDtypeStruct((NC, NPAD, DW), jnp.float32),
        mesh=mesh,
        scratch_types=[
            pltpu.VMEM((B,), jnp.int32),
            pltpu.VMEM((B, D), jnp.float32),
            pltpu.VMEM((B, D), jnp.float32),
            pltpu.VMEM((B, D), jnp.float32),
            pltpu.VMEM((B, DW), jnp.float32),
            pltpu.VMEM_SHARED((NPAD, DW), jnp.float32),
            pltpu.SemaphoreType.DMA,
            pltpu.SemaphoreType.DMA,
            pltpu.SemaphoreType.DMA,
            pltpu.SemaphoreType.DMA,
        ],
        compiler_params=pltpu.CompilerParams(needs_layout_passes=False,
                                             use_tc_tiling_on_sc=False),
    )
    return f(k_hm, v_hm, q_hm, dst4,
             jnp.zeros((ST, DW), jnp.float32),
             jnp.arange(NPAD, dtype=jnp.int32))


def _tc_combine(acc, sel):
    BN = 1280

    def body(a_ref, sel_ref, o_ref):
        p = a_ref[0, :, :D] + a_ref[1, :, :D]
        d = a_ref[0, :, D:] + a_ref[1, :, D:]
        d128 = jnp.dot(d, sel_ref[...], preferred_element_type=jnp.float32)
        o_ref[...] = jnp.where(d128 > 0, p / d128, 0.0)

    return pl.pallas_call(
        body,
        grid=(NPAD // BN,),
        in_specs=[
            pl.BlockSpec((NC, BN, DW), lambda i: (np.int32(0), i, np.int32(0))),
            pl.BlockSpec((16, D), lambda i: (np.int32(0), np.int32(0))),
        ],
        out_specs=pl.BlockSpec((BN, D), lambda i: (i, np.int32(0))),
        out_shape=jax.ShapeDtypeStruct((NPAD, D), jnp.float32),
    )(acc, sel)


_SEL = np.zeros((16, D), np.float32)
for _h in range(N_HEADS):
    _SEL[_h, 16 * _h:16 * _h + 16] = 1.0


def kernel(v0, v1, k0, k1, q0, q1, edge_index):
    E, N, H = N_EDGES, N_NODES, N_HEADS
    npt = NCHUNK // N_TILES
    dst = edge_index[1].astype(jnp.int32).reshape(N_TILES, npt, 1, B)
    one = (edge_index[0, 0] * 0 + 1).astype(jnp.float32)
    K = (jnp.concatenate([k0.reshape(E, H, 4), k1.reshape(E, H, 12)],
                         axis=-1).astype(jnp.float32)
         * (one * np.float32(1.0 / np.sqrt(128.0)))
         ).reshape(N_TILES, npt, B, D)
    V = (jnp.concatenate([v0.reshape(E, H, 4), v1.reshape(E, H, 12)],
                         axis=-1).astype(jnp.float32)
         * one).reshape(N_TILES, npt, B, D)
    Q = jnp.concatenate([q0.reshape(N, H, 4), q1.reshape(N, H, 12)],
                        axis=-1).reshape(N, D).astype(jnp.float32)

    acc = _sc_edge_pass(K, V, Q, dst)
    out128 = _tc_combine(acc, jnp.asarray(_SEL))

    o = out128[:N].reshape(N, H, 16)
    out0 = o[:, :, :4].reshape(N, CH, 1).astype(jnp.float64)
    out1 = o[:, :, 4:].reshape(N, CH, 3).astype(jnp.float64)
    return out0, out1


# speedup vs baseline: 1.0077x; 1.0077x over previous
import numpy as np
import jax
import jax.numpy as jnp
from jax import lax
from jax.experimental import pallas as pl
from jax.experimental.pallas import tpu as pltpu
from jax.experimental.pallas import tpu_sc as plsc

N_NODES = 10000
N_EDGES = 320000
N_HEADS = 8
CH = 32
D = 128
DW = D + 16
B = 40
NCHUNK = N_EDGES // B
NC = 2
NS = 16
N_TILES = NC * NS
NPAD = 10240
ROWS_PER_TILE = NPAD // NS
ST = 40

_K1_ENDS = [(0, 11), (1, 7), (2, 3), (2, 15), (3, 11), (4, 7), (5, 3), (5, 15)]
_V1_SPLIT = [(12, 0, 1), (8, 1, 2), (4, 2, 3), (12, 4, 5), (8, 5, 6), (4, 6, 7)]


def _sc_body(k0_hbm, k1_hbm, v0_hbm, v1_hbm, q_hbm, dst_hbm, z_hbm, rows_hbm,
             accp_hbm, idx_v, qg, kbuf0, kbuf1, vbuf0, vbuf1, msg, acc_sh,
             sem, semq, semk0, semk1, semv0, semv1):
    cid = lax.axis_index("c")
    sid = lax.axis_index("s")
    wid = sid * NC + cid
    r0 = sid * ROWS_PER_TILE
    n_st = ROWS_PER_TILE // ST

    def wrow(j):
        return pl.ds(r0 + ST * j, ST)

    pltpu.sync_copy(z_hbm, msg)
    for j in range(n_st):
        pltpu.sync_copy(rows_hbm.at[wrow(j)], idx_v)
        pltpu.sync_copy(msg, acc_sh.at[idx_v])
    plsc.subcore_barrier()

    lanes = lax.iota(jnp.int32, 16)
    head_mask = lanes < N_HEADS

    n_iter = NCHUNK // N_TILES

    def chunk_body(i, carry):
        cidx = pltpu.async_copy(dst_hbm.at[wid, i, np.int32(0)], idx_v, sem)
        ck0 = pltpu.async_copy(k0_hbm.at[wid, i], kbuf0, semk0)
        ck1 = pltpu.async_copy(k1_hbm.at[wid, i], kbuf1, semk1)
        cv0 = pltpu.async_copy(v0_hbm.at[wid, i], vbuf0, semv0)
        cv1 = pltpu.async_copy(v1_hbm.at[wid, i], vbuf1, semv1)
        cidx.wait()
        cq = pltpu.async_copy(q_hbm.at[idx_v], qg, semq)
        ck0.wait()
        ck1.wait()
        cv0.wait()
        cv1.wait()
        cq.wait()

        def edge_body(e, carry2):
            s = []
            for j in range(2):
                p = kbuf0[e, pl.ds(16 * j, 16)] * qg[e, pl.ds(16 * j, 16)]
                cs = jnp.cumsum(p)
                prev = np.float32(0.0)
                for l in (3, 7, 11, 15):
                    s.append(cs[l] - prev)
                    prev = cs[l]
            cs1 = []
            for j in range(6):
                p = kbuf1[e, pl.ds(16 * j, 16)] * qg[e, pl.ds(32 + 16 * j, 16)]
                cs1.append(jnp.cumsum(p))
            carry_tot = [np.float32(0.0)]
            for j in range(5):
                carry_tot.append(carry_tot[j] + cs1[j][15])
            prev = np.float32(0.0)
            for h, (j, l) in enumerate(_K1_ENDS):
                b = carry_tot[j] + cs1[j][l]
                s[h] = s[h] + (b - prev)
                prev = b
            acc = jnp.zeros((16,), jnp.float32)
            for h in range(N_HEADS):
                acc = jnp.where(lanes == h, s[h], acc)
            w = jnp.where(head_mask, jnp.exp(acc), 0.0)
            for j in range(2):
                m = jnp.where(lanes < 4, w[4 * j],
                              jnp.where(lanes < 8, w[4 * j + 1],
                                        jnp.where(lanes < 12, w[4 * j + 2],
                                                  w[4 * j + 3])))
                msg[e, pl.ds(16 * j, 16)] = vbuf0[e, pl.ds(16 * j, 16)] * m
            for j, (split, ha, hb) in enumerate(_V1_SPLIT):
                m = jnp.where(lanes < split, w[ha], w[hb])
                msg[e, pl.ds(32 + 16 * j, 16)] = \
                    vbuf1[e, pl.ds(16 * j, 16)] * m
            msg[e, pl.ds(D, 16)] = w
            return carry2

        lax.fori_loop(np.int32(0), np.int32(B), edge_body, np.int32(0),
                      unroll=4)

        pltpu.sync_copy(msg, acc_sh.at[idx_v], add=True)
        return carry

    lax.fori_loop(np.int32(0), np.int32(n_iter), chunk_body, np.int32(0))

    plsc.subcore_barrier()
    for j in range(n_st):
        pltpu.sync_copy(rows_hbm.at[wrow(j)], idx_v)
        pltpu.async_copy(acc_sh.at[idx_v], msg, sem).wait()
        pltpu.sync_copy(msg, accp_hbm.at[cid, wrow(j)])


def _sc_edge_pass(k0f, k1f, v0f, v1f, q_f, dst4):
    mesh = plsc.VectorSubcoreMesh(core_axis_name="c", subcore_axis_name="s",
                                  num_cores=NC, num_subcores=NS)
    f = pl.kernel(
        _sc_body,
        out_type=jax.ShapeDtypeStruct((NC, NPAD, DW), jnp.float32),
        mesh=mesh,
        scratch_types=[
            pltpu.VMEM((B,), jnp.int32),
            pltpu.VMEM((B, D), jnp.float32),
            pltpu.VMEM((B, 32), jnp.float32),
            pltpu.VMEM((B, 96), jnp.float32),
            pltpu.VMEM((B, 32), jnp.float32),
            pltpu.VMEM((B, 96), jnp.float32),
            pltpu.VMEM((B, DW), jnp.float32),
            pltpu.VMEM_SHARED((NPAD, DW), jnp.float32),
            pltpu.SemaphoreType.DMA,
            pltpu.SemaphoreType.DMA,
            pltpu.SemaphoreType.DMA,
            pltpu.SemaphoreType.DMA,
            pltpu.SemaphoreType.DMA,
            pltpu.SemaphoreType.DMA,
        ],
        compiler_params=pltpu.CompilerParams(needs_layout_passes=False,
                                             use_tc_tiling_on_sc=False),
    )
    return f(k0f, k1f, v0f, v1f, q_f, dst4,
             jnp.zeros((ST, DW), jnp.float32),
             jnp.arange(NPAD, dtype=jnp.int32))


def _tc_combine(acc, sel):
    BN = 1280

    def body(a_ref, sel_ref, o_ref):
        p = a_ref[0, :, :D] + a_ref[1, :, :D]
        d = a_ref[0, :, D:] + a_ref[1, :, D:]
        d128 = jnp.dot(d, sel_ref[...], preferred_element_type=jnp.float32)
        o_ref[...] = jnp.where(d128 > 0, p / d128, 0.0)

    return pl.pallas_call(
        body,
        grid=(NPAD // BN,),
        in_specs=[
            pl.BlockSpec((NC, BN, DW), lambda i: (np.int32(0), i, np.int32(0))),
            pl.BlockSpec((16, D), lambda i: (np.int32(0), np.int32(0))),
        ],
        out_specs=pl.BlockSpec((BN, D), lambda i: (i, np.int32(0))),
        out_shape=jax.ShapeDtypeStruct((NPAD, D), jnp.float32),
    )(acc, sel)


_SEL = np.zeros((16, D), np.float32)
for _c in range(D):
    _SEL[_c // 4 if _c < 32 else (_c - 32) // 12, _c] = 1.0


def kernel(v0, v1, k0, k1, q0, q1, edge_index):
    E, N, H = N_EDGES, N_NODES, N_HEADS
    npt = NCHUNK // N_TILES
    dst = edge_index[1].astype(jnp.int32).reshape(N_TILES, npt, 1, B)
    k0f = k0.astype(jnp.float32).reshape(N_TILES, npt, B, 32)
    k1f = k1.astype(jnp.float32).reshape(N_TILES, npt, B, 96)
    v0f = v0.astype(jnp.float32).reshape(N_TILES, npt, B, 32)
    v1f = v1.astype(jnp.float32).reshape(N_TILES, npt, B, 96)
    Q = (jnp.concatenate([q0.reshape(N, 32), q1.reshape(N, 96)], axis=-1)
         .astype(jnp.float32) * np.float32(1.0 / np.sqrt(128.0)))

    acc = _sc_edge_pass(k0f, k1f, v0f, v1f, Q, dst)
    out128 = _tc_combine(acc, jnp.asarray(_SEL))

    out0 = out128[:N, :32].reshape(N, CH, 1).astype(jnp.float64)
    out1 = out128[:N, 32:].reshape(N, CH, 3).astype(jnp.float64)
    return out0, out1
